# speedup vs baseline: 10.7042x; 10.7042x over previous
import functools

import jax
import jax.numpy as jnp
from jax import lax
from jax.experimental import pallas as pl

B, N, NS, D, S = 4, 2048, 16, 128, 8
EPS = 1e-5
RB = 256
PB = 512
NBLK = N // RB


def _dotT(a, b):
    return lax.dot_general(a, b, (((1,), (1,)), ((), ())),
                           preferred_element_type=jnp.float32)


def _proj_kernel(t0_ref, lin1_W_ref, bn1_gs_ref, bn1_b_ref,
                 q_W_ref, q_b_ref, k_W_ref, k_b_ref, v_W_ref, v_b_ref,
                 xq_ref, kv_ref):
    t0 = t0_ref[...]
    x = jnp.maximum(_dotT(t0, lin1_W_ref[...]) * bn1_gs_ref[...]
                    + bn1_b_ref[...], 0.0)
    xq = _dotT(x, q_W_ref[...]) + q_b_ref[...]
    xk = _dotT(x, k_W_ref[...]) + k_b_ref[...]
    xv = _dotT(x, v_W_ref[...]) + v_b_ref[...]
    xq_ref[...] = xq
    kv_ref[...] = jnp.concatenate([xk, xv], axis=1)


def _attn_kernel(p_blk_ref, p_full_ref, xq_ref, t0_ref, kv_ref,
                 p0_W_ref, p0_b_ref, pbn_gs_ref, pbn_b_ref,
                 p2_W_ref, p2_b_ref, wbn0_gs_ref, wbn0_b_ref,
                 w2_W_ref, w2_b_ref, wbn3_gs_ref, wbn3_b_ref,
                 w5_W_ref, w5_b_ref, bn2_gs_ref, bn2_b_ref,
                 lin3_W_ref, bn3_gs_ref, bn3_b_ref,
                 mlp1_W_ref, mlp1_b_ref, mlpbn_gs_ref, mlpbn_b_ref,
                 mlp2_W_ref, out_ref):
    p_blk = p_blk_ref[...]
    p_full = p_full_ref[...]
    xq = xq_ref[...]
    kv = kv_ref[...]

    sq_blk = jnp.sum(p_blk * p_blk, axis=1, keepdims=True)
    sq_full = jnp.sum(p_full * p_full, axis=1, keepdims=True)
    dot = _dotT(p_blk, p_full)
    d2 = sq_blk + jnp.transpose(sq_full) - 2.0 * dot

    iota = lax.broadcasted_iota(jnp.int32, (RB, N), 1)

    w_list = []
    v_list = []
    for _ in range(NS):
        m = jnp.min(d2, axis=1, keepdims=True)
        idx = jnp.min(jnp.where(d2 == m, iota, N), axis=1, keepdims=True)
        onehot = iota == idx
        d2 = jnp.where(onehot, jnp.inf, d2)
        oh_f = onehot.astype(jnp.float32)
        g_kv = lax.dot_general(oh_f, kv, (((1,), (0,)), ((), ())),
                               preferred_element_type=jnp.float32)
        g_p = lax.dot_general(oh_f, p_full, (((1,), (0,)), ((), ())),
                              preferred_element_type=jnp.float32)
        xk_s = g_kv[:, :D]
        xv_s = g_kv[:, D:]
        p_r = g_p - p_blk
        pr = _dotT(p_r, p0_W_ref[...]) + p0_b_ref[...]
        pr = jnp.maximum(pr * pbn_gs_ref[...] + pbn_b_ref[...], 0.0)
        pr = _dotT(pr, p2_W_ref[...]) + p2_b_ref[...]
        w = xk_s - xq + pr
        w = jnp.maximum(w * wbn0_gs_ref[...] + wbn0_b_ref[...], 0.0)
        w = _dotT(w, w2_W_ref[...]) + w2_b_ref[...]
        w = jnp.maximum(w * wbn3_gs_ref[...] + wbn3_b_ref[...], 0.0)
        w = _dotT(w, w5_W_ref[...]) + w5_b_ref[...]
        w_list.append(w)
        v_list.append(xv_s + pr)

    mx = w_list[0]
    for w in w_list[1:]:
        mx = jnp.maximum(mx, w)
    e_list = [jnp.exp(w - mx) for w in w_list]
    z = e_list[0]
    for e in e_list[1:]:
        z = z + e
    rz = 1.0 / z

    c_idx = lax.broadcasted_iota(jnp.int32, (D // S, D), 1)
    g_idx = lax.broadcasted_iota(jnp.int32, (D // S, D), 0)
    expand = (jnp.remainder(c_idx, D // S) == g_idx).astype(jnp.float32)

    attn = jnp.zeros((RB, D), dtype=jnp.float32)
    for e, v in zip(e_list, v_list):
        wt = lax.dot_general(e * rz, expand, (((1,), (0,)), ((), ())),
                             preferred_element_type=jnp.float32)
        attn = attn + wt * v

    x2 = jnp.maximum(attn * bn2_gs_ref[...] + bn2_b_ref[...], 0.0)
    x3 = _dotT(x2, lin3_W_ref[...]) * bn3_gs_ref[...] + bn3_b_ref[...]
    x4 = jnp.maximum(x3 + t0_ref[...], 0.0)
    h = jnp.maximum((_dotT(x4, mlp1_W_ref[...]) + mlp1_b_ref[...])
                    * mlpbn_gs_ref[...] + mlpbn_b_ref[...], 0.0)
    y = _dotT(h, mlp2_W_ref[...])
    out_ref[...] = p_blk + y


def kernel(pxo, transf_features, idxs, lin1_W, bn1_g, bn1_b, q_W, q_b,
           k_W, k_b, v_W, v_b, p0_W, p0_b, pbn_g, pbn_b, p2_W, p2_b,
           wbn0_g, wbn0_b, w2_W, w2_b, wbn3_g, wbn3_b, w5_W, w5_b,
           bn2_g, bn2_b, lin3_W, bn3_g, bn3_b, mlp1_W, mlp1_b,
           mlpbn_g, mlpbn_b, mlp2_W):
    n = B * N
    pxo = pxo + jnp.sum(idxs).astype(pxo.dtype)
    p_flat = pxo.reshape(n, 3)
    t0 = jnp.transpose(transf_features, (0, 2, 1)).reshape(n, D)

    s = 1.0 / jnp.sqrt(jnp.float32(1.0 + EPS))
    r1 = lambda a: a.reshape(1, -1)
    bn1_gs = r1(bn1_g * s); bn1_b2 = r1(bn1_b)
    pbn_gs = r1(pbn_g * s); pbn_b2 = r1(pbn_b)
    wbn0_gs = r1(wbn0_g * s); wbn0_b2 = r1(wbn0_b)
    wbn3_gs = r1(wbn3_g * s); wbn3_b2 = r1(wbn3_b)
    bn2_gs = r1(bn2_g * s); bn2_b2 = r1(bn2_b)
    bn3_gs = r1(bn3_g * s); bn3_b2 = r1(bn3_b)
    mlpbn_gs = r1(mlpbn_g * s); mlpbn_b2 = r1(mlpbn_b)
    q_b2 = r1(q_b); k_b2 = r1(k_b); v_b2 = r1(v_b)
    p0_b2 = r1(p0_b); p2_b2 = r1(p2_b); w2_b2 = r1(w2_b); w5_b2 = r1(w5_b)
    mlp1_b2 = r1(mlp1_b)

    fullb = lambda shp: pl.BlockSpec(shp, lambda b, i: (0,) * len(shp))

    pwts = (lin1_W, bn1_gs, bn1_b2, q_W, q_b2, k_W, k_b2, v_W, v_b2)
    xq, kv = pl.pallas_call(
        _proj_kernel,
        grid=(n // PB,),
        in_specs=[pl.BlockSpec((PB, D), lambda i: (i, 0))]
        + [pl.BlockSpec(w.shape, lambda i: (0, 0)) for w in pwts],
        out_specs=[pl.BlockSpec((PB, D), lambda i: (i, 0)),
                   pl.BlockSpec((PB, 2 * D), lambda i: (i, 0))],
        out_shape=[jax.ShapeDtypeStruct((n, D), jnp.float32),
                   jax.ShapeDtypeStruct((n, 2 * D), jnp.float32)],
    )(t0, *pwts)

    wts = (p0_W, p0_b2, pbn_gs, pbn_b2, p2_W, p2_b2, wbn0_gs, wbn0_b2,
           w2_W, w2_b2, wbn3_gs, wbn3_b2, w5_W, w5_b2, bn2_gs, bn2_b2,
           lin3_W, bn3_gs, bn3_b2, mlp1_W, mlp1_b2, mlpbn_gs, mlpbn_b2,
           mlp2_W)

    out = pl.pallas_call(
        _attn_kernel,
        grid=(B, NBLK),
        in_specs=[
            pl.BlockSpec((RB, 3), lambda b, i: (b * NBLK + i, 0)),
            pl.BlockSpec((N, 3), lambda b, i: (b, 0)),
            pl.BlockSpec((RB, D), lambda b, i: (b * NBLK + i, 0)),
            pl.BlockSpec((RB, D), lambda b, i: (b * NBLK + i, 0)),
            pl.BlockSpec((N, 2 * D), lambda b, i: (b, 0)),
        ] + [fullb(w.shape) for w in wts],
        out_specs=pl.BlockSpec((RB, 3), lambda b, i: (b * NBLK + i, 0)),
        out_shape=jax.ShapeDtypeStruct((n, 3), jnp.float32),
    )(p_flat, p_flat, xq, t0, kv, *wts)

    return jnp.transpose(out.reshape(B, N, 3), (0, 2, 1))


# speedup vs baseline: 11.8558x; 1.1076x over previous
import functools

import jax
import jax.numpy as jnp
from jax import lax
from jax.experimental import pallas as pl

B, N, NS, D, S = 4, 2048, 16, 128, 8
EPS = 1e-5
RB = 256
PB = 512
NBLK = N // RB


def _dotT(a, b):
    return lax.dot_general(a, b, (((1,), (1,)), ((), ())),
                           preferred_element_type=jnp.float32)


def _proj_kernel(t0_ref, ppad_ref, lin1_W_ref, bn1_gs_ref, bn1_b_ref,
                 q_W_ref, q_b_ref, k_W_ref, k_b_ref, v_W_ref, v_b_ref,
                 xq_ref, kvp_ref):
    t0 = t0_ref[...]
    x = jnp.maximum(_dotT(t0, lin1_W_ref[...]) * bn1_gs_ref[...]
                    + bn1_b_ref[...], 0.0)
    xq = _dotT(x, q_W_ref[...]) + q_b_ref[...]
    xk = _dotT(x, k_W_ref[...]) + k_b_ref[...]
    xv = _dotT(x, v_W_ref[...]) + v_b_ref[...]
    xq_ref[...] = xq
    kvp_ref[...] = jnp.concatenate(
        [xk, xv, ppad_ref[...]], axis=1).astype(jnp.bfloat16)


def _attn_kernel(p_blk_ref, p_full_ref, xq_ref, t0_ref, kvp_ref,
                 p0_W_ref, p0_b_ref, pbn_gs_ref, pbn_b_ref,
                 p2_W_ref, p2_b_ref, wbn0_gs_ref, wbn0_b_ref,
                 w2_W_ref, w2_b_ref, wbn3_gs_ref, wbn3_b_ref,
                 w5_W_ref, w5_b_ref, bn2_gs_ref, bn2_b_ref,
                 lin3_W_ref, bn3_gs_ref, bn3_b_ref,
                 mlp1_W_ref, mlp1_b_ref, mlpbn_gs_ref, mlpbn_b_ref,
                 mlp2_W_ref, out_ref):
    p_blk = p_blk_ref[...]
    p_full = p_full_ref[...]
    xq = xq_ref[...]
    kvp = kvp_ref[...]

    sq_blk = jnp.sum(p_blk * p_blk, axis=1, keepdims=True)
    sq_full = jnp.sum(p_full * p_full, axis=1, keepdims=True)
    dot = _dotT(p_blk, p_full)
    d2 = sq_blk + jnp.transpose(sq_full) - 2.0 * dot

    w_list = []
    v_list = []
    for _ in range(NS):
        m = jnp.min(d2, axis=1, keepdims=True)
        eq = d2 == m
        oh_b = jnp.where(eq, 1.0, 0.0).astype(jnp.bfloat16)
        d2 = jnp.where(eq, jnp.inf, d2)
        g = lax.dot_general(oh_b, kvp, (((1,), (0,)), ((), ())),
                            preferred_element_type=jnp.float32)
        xk_s = g[:, :D]
        xv_s = g[:, D:2 * D]
        p_r = g[:, 2 * D:2 * D + 3] - p_blk
        pr = _dotT(p_r, p0_W_ref[...]) + p0_b_ref[...]
        pr = jnp.maximum(pr * pbn_gs_ref[...] + pbn_b_ref[...], 0.0)
        pr = _dotT(pr, p2_W_ref[...]) + p2_b_ref[...]
        w = xk_s - xq + pr
        w = jnp.maximum(w * wbn0_gs_ref[...] + wbn0_b_ref[...], 0.0)
        w = _dotT(w, w2_W_ref[...]) + w2_b_ref[...]
        w = jnp.maximum(w * wbn3_gs_ref[...] + wbn3_b_ref[...], 0.0)
        w = _dotT(w, w5_W_ref[...]) + w5_b_ref[...]
        w_list.append(w)
        v_list.append(xv_s + pr)

    mx = w_list[0]
    for w in w_list[1:]:
        mx = jnp.maximum(mx, w)
    e_list = [jnp.exp(w - mx) for w in w_list]
    z = e_list[0]
    for e in e_list[1:]:
        z = z + e
    rz = 1.0 / z

    c_idx = lax.broadcasted_iota(jnp.int32, (D // S, D), 1)
    g_idx = lax.broadcasted_iota(jnp.int32, (D // S, D), 0)
    expand = (jnp.remainder(c_idx, D // S) == g_idx).astype(jnp.float32)

    attn = jnp.zeros((RB, D), dtype=jnp.float32)
    for e, v in zip(e_list, v_list):
        wt = lax.dot_general(e * rz, expand, (((1,), (0,)), ((), ())),
                             preferred_element_type=jnp.float32)
        attn = attn + wt * v

    x2 = jnp.maximum(attn * bn2_gs_ref[...] + bn2_b_ref[...], 0.0)
    x3 = _dotT(x2, lin3_W_ref[...]) * bn3_gs_ref[...] + bn3_b_ref[...]
    x4 = jnp.maximum(x3 + t0_ref[...], 0.0)
    h = jnp.maximum((_dotT(x4, mlp1_W_ref[...]) + mlp1_b_ref[...])
                    * mlpbn_gs_ref[...] + mlpbn_b_ref[...], 0.0)
    y = _dotT(h, mlp2_W_ref[...])
    out_ref[...] = p_blk + y


def kernel(pxo, transf_features, idxs, lin1_W, bn1_g, bn1_b, q_W, q_b,
           k_W, k_b, v_W, v_b, p0_W, p0_b, pbn_g, pbn_b, p2_W, p2_b,
           wbn0_g, wbn0_b, w2_W, w2_b, wbn3_g, wbn3_b, w5_W, w5_b,
           bn2_g, bn2_b, lin3_W, bn3_g, bn3_b, mlp1_W, mlp1_b,
           mlpbn_g, mlpbn_b, mlp2_W):
    n = B * N
    pxo = pxo + jnp.sum(idxs).astype(pxo.dtype)
    p_flat = pxo.reshape(n, 3)
    t0 = jnp.transpose(transf_features, (0, 2, 1)).reshape(n, D)

    s = 1.0 / jnp.sqrt(jnp.float32(1.0 + EPS))
    r1 = lambda a: a.reshape(1, -1)
    bn1_gs = r1(bn1_g * s); bn1_b2 = r1(bn1_b)
    pbn_gs = r1(pbn_g * s); pbn_b2 = r1(pbn_b)
    wbn0_gs = r1(wbn0_g * s); wbn0_b2 = r1(wbn0_b)
    wbn3_gs = r1(wbn3_g * s); wbn3_b2 = r1(wbn3_b)
    bn2_gs = r1(bn2_g * s); bn2_b2 = r1(bn2_b)
    bn3_gs = r1(bn3_g * s); bn3_b2 = r1(bn3_b)
    mlpbn_gs = r1(mlpbn_g * s); mlpbn_b2 = r1(mlpbn_b)
    q_b2 = r1(q_b); k_b2 = r1(k_b); v_b2 = r1(v_b)
    p0_b2 = r1(p0_b); p2_b2 = r1(p2_b); w2_b2 = r1(w2_b); w5_b2 = r1(w5_b)
    mlp1_b2 = r1(mlp1_b)

    fullb = lambda shp: pl.BlockSpec(shp, lambda b, i: (0,) * len(shp))

    ppad = jnp.pad(p_flat, ((0, 0), (0, D - 3)))

    pwts = (lin1_W, bn1_gs, bn1_b2, q_W, q_b2, k_W, k_b2, v_W, v_b2)
    xq, kvp = pl.pallas_call(
        _proj_kernel,
        grid=(n // PB,),
        in_specs=[pl.BlockSpec((PB, D), lambda i: (i, 0)),
                  pl.BlockSpec((PB, D), lambda i: (i, 0))]
        + [pl.BlockSpec(w.shape, lambda i: (0, 0)) for w in pwts],
        out_specs=[pl.BlockSpec((PB, D), lambda i: (i, 0)),
                   pl.BlockSpec((PB, 3 * D), lambda i: (i, 0))],
        out_shape=[jax.ShapeDtypeStruct((n, D), jnp.float32),
                   jax.ShapeDtypeStruct((n, 3 * D), jnp.bfloat16)],
    )(t0, ppad, *pwts)

    wts = (p0_W, p0_b2, pbn_gs, pbn_b2, p2_W, p2_b2, wbn0_gs, wbn0_b2,
           w2_W, w2_b2, wbn3_gs, wbn3_b2, w5_W, w5_b2, bn2_gs, bn2_b2,
           lin3_W, bn3_gs, bn3_b2, mlp1_W, mlp1_b2, mlpbn_gs, mlpbn_b2,
           mlp2_W)

    out = pl.pallas_call(
        _attn_kernel,
        grid=(B, NBLK),
        in_specs=[
            pl.BlockSpec((RB, 3), lambda b, i: (b * NBLK + i, 0)),
            pl.BlockSpec((N, 3), lambda b, i: (b, 0)),
            pl.BlockSpec((RB, D), lambda b, i: (b * NBLK + i, 0)),
            pl.BlockSpec((RB, D), lambda b, i: (b * NBLK + i, 0)),
            pl.BlockSpec((N, 3 * D), lambda b, i: (b, 0)),
        ] + [fullb(w.shape) for w in wts],
        out_specs=pl.BlockSpec((RB, 3), lambda b, i: (b * NBLK + i, 0)),
        out_shape=jax.ShapeDtypeStruct((n, 3), jnp.float32),
    )(p_flat, p_flat, xq, t0, kvp, *wts)

    return jnp.transpose(out.reshape(B, N, 3), (0, 2, 1))
